# baseline (speedup 1.0000x reference)
import functools

import jax
import jax.numpy as jnp
from jax import lax
from jax.experimental import pallas as pl
from jax.experimental.pallas import tpu as pltpu
from jax.experimental.pallas import tpu_sc as plsc

_B = 16384
_D = 64
_CHUNK = 256


def kernel(user_table, item_table, user_idx, item_idx):
    info = plsc.get_sparse_core_info()
    nw = info.num_cores * info.num_subcores
    bpw = _B // nw
    nchunk = bpw // _CHUNK

    mesh = plsc.VectorSubcoreMesh(core_axis_name="c", subcore_axis_name="s")

    @functools.partial(
        pl.kernel,
        mesh=mesh,
        out_type=jax.ShapeDtypeStruct((2, _D, _B), jnp.float32),
        scratch_types=[
            pltpu.VMEM((bpw,), jnp.int32),
            pltpu.VMEM((bpw,), jnp.int32),
            pltpu.VMEM((_CHUNK, _D), jnp.float32),
            pltpu.VMEM((_CHUNK, _D), jnp.float32),
            pltpu.VMEM((_CHUNK * _D,), jnp.float32),
            pltpu.SemaphoreType.DMA,
            pltpu.SemaphoreType.DMA,
            pltpu.SemaphoreType.DMA,
        ],
        compiler_params=pltpu.CompilerParams(use_tc_tiling_on_sc=True,
                                             needs_layout_passes=False),
    )
    def _emb(ut, it, ui, ii, out, uidx_v, iidx_v, ubuf, ibuf, tbuf,
             usem, isem, wsem):
        wid = lax.axis_index("s") * info.num_cores + lax.axis_index("c")
        base = wid * bpw
        pltpu.sync_copy(ui.at[pl.ds(base, bpw)], uidx_v)
        pltpu.sync_copy(ii.at[pl.ds(base, bpw)], iidx_v)

        def enqueue(tbl, idx_v, buf, sem, c):
            def body(g, carry):
                vec = idx_v[pl.ds(c * _CHUNK + g * 16, 16)]
                for k in range(16):
                    pltpu.async_copy(tbl.at[vec[k]], buf.at[g * 16 + k], sem)
                return carry
            lax.fori_loop(0, _CHUNK // 16, body, 0)

        lanes = jnp.arange(16, dtype=jnp.int32)

        def transpose_chunk(buf):
            def body(g, carry):
                rows = g * 16 + lanes
                for d in range(_D):
                    cols = jnp.full((16,), d, dtype=jnp.int32)
                    tbuf[pl.ds(d * _CHUNK + g * 16, 16)] = (
                        plsc.load_gather(buf, [rows, cols]))
                return carry
            lax.fori_loop(0, _CHUNK // 16, body, 0)

        def write_out(t, c):
            col = base + c * _CHUNK
            col = pl.multiple_of(col, 128)
            for d in range(_D):
                pltpu.async_copy(tbuf.at[pl.ds(d * _CHUNK, _CHUNK)],
                                 out.at[t, d].at[pl.ds(col, _CHUNK)], wsem)
            pltpu.make_async_copy(out.at[0, 0], tbuf, wsem).wait()

        enqueue(ut, uidx_v, ubuf, usem, 0)
        enqueue(it, iidx_v, ibuf, isem, 0)
        for c in range(nchunk):
            pltpu.make_async_copy(ut.at[pl.ds(0, _CHUNK)], ubuf, usem).wait()
            transpose_chunk(ubuf)
            if c + 1 < nchunk:
                enqueue(ut, uidx_v, ubuf, usem, c + 1)
            write_out(0, c)
            pltpu.make_async_copy(it.at[pl.ds(0, _CHUNK)], ibuf, isem).wait()
            transpose_chunk(ibuf)
            if c + 1 < nchunk:
                enqueue(it, iidx_v, ibuf, isem, c + 1)
            write_out(1, c)

    out = _emb(user_table, item_table,
               user_idx.astype(jnp.int32), item_idx.astype(jnp.int32))
    return out.transpose(0, 2, 1)


# speedup vs baseline: 1.2333x; 1.2333x over previous
import functools

import jax
import jax.numpy as jnp
from jax import lax
from jax.experimental import pallas as pl
from jax.experimental.pallas import tpu as pltpu
from jax.experimental.pallas import tpu_sc as plsc

_B = 16384
_D = 64
_CHUNK = 256


def _make_gather():
    info = plsc.get_sparse_core_info()
    nw = info.num_cores * info.num_subcores
    bpw = _B // nw

    mesh = plsc.VectorSubcoreMesh(core_axis_name="c", subcore_axis_name="s")

    @functools.partial(
        pl.kernel,
        mesh=mesh,
        out_type=jax.ShapeDtypeStruct((_B, _D), jnp.float32),
        scratch_types=[
            pltpu.VMEM((bpw,), jnp.int32),
            pltpu.VMEM((_CHUNK, _D), jnp.float32),
            pltpu.VMEM((_CHUNK, _D), jnp.float32),
            pltpu.SemaphoreType.DMA,
            pltpu.SemaphoreType.DMA,
        ],
        compiler_params=pltpu.CompilerParams(use_tc_tiling_on_sc=True),
    )
    def _gather(tbl, idx, out, idx_v, buf_a, buf_b, sem_a, sem_b):
        wid = lax.axis_index("s") * info.num_cores + lax.axis_index("c")
        base = wid * bpw
        pltpu.sync_copy(idx.at[pl.ds(base, bpw)], idx_v)

        def enqueue(buf, sem, c):
            def body(g, carry):
                vec = idx_v[pl.ds(c * _CHUNK + g * 16, 16)]
                for k in range(16):
                    pltpu.async_copy(tbl.at[vec[k]], buf.at[g * 16 + k], sem)
                return carry
            lax.fori_loop(0, _CHUNK // 16, body, 0)

        enqueue(buf_a, sem_a, 0)
        enqueue(buf_b, sem_b, 1)
        pltpu.make_async_copy(tbl.at[pl.ds(0, _CHUNK)], buf_a, sem_a).wait()
        pltpu.sync_copy(buf_a, out.at[pl.ds(base, _CHUNK)])
        pltpu.make_async_copy(tbl.at[pl.ds(0, _CHUNK)], buf_b, sem_b).wait()
        pltpu.sync_copy(buf_b, out.at[pl.ds(base + _CHUNK, _CHUNK)])

    return _gather


def kernel(user_table, item_table, user_idx, item_idx):
    gather = _make_gather()
    user_x = gather(user_table, user_idx.astype(jnp.int32))
    item_x = gather(item_table, item_idx.astype(jnp.int32))
    return jnp.concatenate([user_x, item_x], axis=0).reshape(2, _B, _D)


# speedup vs baseline: 1.2972x; 1.0519x over previous
import functools

import jax
import jax.numpy as jnp
from jax import lax
from jax.experimental import pallas as pl
from jax.experimental.pallas import tpu as pltpu
from jax.experimental.pallas import tpu_sc as plsc

_B = 16384
_D = 64
_CHUNK = 128
_NBUF = 2


def kernel(user_table, item_table, user_idx, item_idx):
    info = plsc.get_sparse_core_info()
    nw = info.num_cores * info.num_subcores
    bpw = _B // nw
    nchunk = bpw // _CHUNK

    mesh = plsc.VectorSubcoreMesh(core_axis_name="c", subcore_axis_name="s")

    @functools.partial(
        pl.kernel,
        mesh=mesh,
        out_type=jax.ShapeDtypeStruct((2, _B, _D), jnp.float32),
        scratch_types=[
            pltpu.VMEM((bpw,), jnp.int32),
            pltpu.VMEM((bpw,), jnp.int32),
            pltpu.VMEM((_NBUF, _CHUNK, _D), jnp.float32),
            pltpu.VMEM((_NBUF, _CHUNK, _D), jnp.float32),
            pltpu.SemaphoreType.DMA((2, _NBUF)),
        ],
        compiler_params=pltpu.CompilerParams(use_tc_tiling_on_sc=True),
    )
    def _emb(ut, it, ui, ii, out, uidx_v, iidx_v, ubufs, ibufs, sems):
        wid = lax.axis_index("s") * info.num_cores + lax.axis_index("c")
        base = wid * bpw
        pltpu.sync_copy(ui.at[pl.ds(base, bpw)], uidx_v)
        pltpu.sync_copy(ii.at[pl.ds(base, bpw)], iidx_v)

        tables = ((ut, uidx_v, ubufs, 0), (it, iidx_v, ibufs, 1))

        def enqueue(t, c):
            tbl, idx_v, bufs, _ = tables[t]
            buf = bufs.at[c % _NBUF]
            sem = sems.at[t, c % _NBUF]

            def body(g, carry):
                vec = idx_v[pl.ds(c * _CHUNK + g * 16, 16)]
                for k in range(16):
                    pltpu.async_copy(tbl.at[vec[k]], buf.at[g * 16 + k], sem)
                return carry
            lax.fori_loop(0, _CHUNK // 16, body, 0)

        for b in range(_NBUF):
            enqueue(0, b)
            enqueue(1, b)
        for c in range(nchunk):
            for t in range(2):
                tbl, _, bufs, slot = tables[t]
                buf = bufs.at[c % _NBUF]
                pltpu.make_async_copy(tbl.at[pl.ds(0, _CHUNK)], buf,
                                      sems.at[t, c % _NBUF]).wait()
                pltpu.sync_copy(buf,
                                out.at[slot, pl.ds(base + c * _CHUNK, _CHUNK)])
                if c + _NBUF < nchunk:
                    enqueue(t, c + _NBUF)

    return _emb(user_table, item_table,
                user_idx.astype(jnp.int32), item_idx.astype(jnp.int32))
